# baseline (speedup 1.0000x reference)
import functools

import jax
import jax.numpy as jnp
from jax import lax
from jax.experimental import pallas as pl
from jax.experimental.pallas import tpu as pltpu
from jax.experimental.pallas import tpu_sc as plsc

_NC = 2
_NS = 16
_NW = _NC * _NS
_L = 16
_EB = 16
_D = 64


def _rsqrt_newton(x):
    x = jnp.maximum(x, jnp.float32(1e-24))
    i = lax.bitcast_convert_type(x, jnp.int32)
    i = jnp.int32(0x5F3759DF) - lax.shift_right_arithmetic(i, 1)
    y = lax.bitcast_convert_type(i, jnp.float32)
    for _ in range(3):
        y = y * (jnp.float32(1.5) - jnp.float32(0.5) * x * y * y)
    return y


def kernel(h, r, t, entity_embed, relation_embed):
    B = h.shape[0]
    N = entity_embed.shape[0]
    D = entity_embed.shape[1]
    assert D == _D and N % _L == 0
    bpw = B // _NW
    nblk = bpw // _EB
    ecpd = N // _L
    _RP = 1024
    rcpd = _RP // _L
    nrow = _EB * D

    ent16 = entity_embed.T.reshape(N * D // _L, _L)
    relp = jnp.pad(relation_embed.T, ((0, 0), (0, _RP - relation_embed.shape[0])))
    rel16 = relp.reshape(_RP * D // _L, _L)

    h2d = h.reshape(_NW, bpw)
    r2d = r.reshape(_NW, bpw)
    t2d = t.reshape(_NW, bpw)

    mesh = plsc.VectorSubcoreMesh(core_axis_name="c", subcore_axis_name="s")

    @functools.partial(
        pl.kernel,
        out_type=jax.ShapeDtypeStruct((B,), jnp.float32),
        mesh=mesh,
        compiler_params=pltpu.CompilerParams(
            needs_layout_passes=False, use_tc_tiling_on_sc=False),
        scratch_types=[
            pltpu.VMEM((bpw,), jnp.int32),
            pltpu.VMEM((bpw,), jnp.int32),
            pltpu.VMEM((bpw,), jnp.int32),
            pltpu.VMEM((nrow,), jnp.int32),
            pltpu.VMEM((nrow,), jnp.int32),
            pltpu.VMEM((nrow,), jnp.int32),
            pltpu.VMEM((nrow, _L), jnp.float32),
            pltpu.VMEM((nrow, _L), jnp.float32),
            pltpu.VMEM((nrow, _L), jnp.float32),
            pltpu.VMEM((bpw,), jnp.float32),
            pltpu.SemaphoreType.DMA,
        ],
    )
    def run(h_hbm, r_hbm, t_hbm, ent_hbm, rel_hbm, out_hbm,
            hi, ri, ti, hx, rx, tx, hb, rb, tb, ov, sem):
        wid = lax.axis_index("s") * _NC + lax.axis_index("c")

        pltpu.sync_copy(h_hbm.at[wid], hi)
        pltpu.sync_copy(r_hbm.at[wid], ri)
        pltpu.sync_copy(t_hbm.at[wid], ti)

        lanes = lax.iota(jnp.int32, _L)

        def block(b, carry):
            base = pl.multiple_of(b * _EB, _EB)
            he = hi[pl.ds(base, _L)]
            re = ri[pl.ds(base, _L)]
            te = ti[pl.ds(base, _L)]

            hc = lax.shift_right_arithmetic(he, 4)
            rc = lax.shift_right_arithmetic(re, 4)
            tc = lax.shift_right_arithmetic(te, 4)
            for j in range(_D):
                hx[pl.ds(j * _EB, _L)] = hc + jnp.int32(j * ecpd)
                rx[pl.ds(j * _EB, _L)] = rc + jnp.int32(j * rcpd)
                tx[pl.ds(j * _EB, _L)] = tc + jnp.int32(j * ecpd)

            cph = pltpu.async_copy(ent_hbm.at[hx], hb, sem)
            cpr = pltpu.async_copy(rel_hbm.at[rx], rb, sem)
            cpt = pltpu.async_copy(ent_hbm.at[tx], tb, sem)
            cph.wait()
            cpr.wait()
            cpt.wait()

            ho = he & jnp.int32(_L - 1)
            ro = re & jnp.int32(_L - 1)
            to = te & jnp.int32(_L - 1)

            h2 = jnp.zeros((_L,), jnp.float32)
            t2 = jnp.zeros((_L,), jnp.float32)
            for j in range(_D):
                rows = lanes + jnp.int32(j * _EB)
                hj = plsc.load_gather(hb, [rows, ho])
                tj = plsc.load_gather(tb, [rows, to])
                h2 = h2 + hj * hj
                t2 = t2 + tj * tj
            ih = _rsqrt_newton(h2)
            it = _rsqrt_newton(t2)

            d = jnp.zeros((_L,), jnp.float32)
            for j in range(_D):
                rows = lanes + jnp.int32(j * _EB)
                hj = plsc.load_gather(hb, [rows, ho])
                rj = plsc.load_gather(rb, [rows, ro])
                tj = plsc.load_gather(tb, [rows, to])
                d = d + jnp.abs(hj * ih + rj - tj * it)
            ov[pl.ds(base, _L)] = -d
            return carry

        lax.fori_loop(0, nblk, block, 0)
        pltpu.sync_copy(ov, out_hbm.at[pl.ds(wid * bpw, bpw)])

    return run(h2d, r2d, t2d, ent16, rel16)


# speedup vs baseline: 3.4377x; 3.4377x over previous
import functools

import jax
import jax.numpy as jnp
from jax import lax
from jax.experimental import pallas as pl
from jax.experimental.pallas import tpu as pltpu
from jax.experimental.pallas import tpu_sc as plsc

_NC = 2
_NS = 16
_NW = _NC * _NS
_L = 16
_CHUNK = 128


def _rsqrt_newton(x):
    x = jnp.maximum(x, jnp.float32(1e-24))
    i = lax.bitcast_convert_type(x, jnp.int32)
    i = jnp.int32(0x5F3759DF) - lax.shift_right_arithmetic(i, 1)
    y = lax.bitcast_convert_type(i, jnp.float32)
    for _ in range(3):
        y = y * (jnp.float32(1.5) - jnp.float32(0.5) * x * y * y)
    return y


def kernel(h, r, t, entity_embed, relation_embed):
    B = h.shape[0]
    D = entity_embed.shape[1]
    bpw = B // _NW
    nch = bpw // _CHUNK
    ngrp = bpw // _L

    h3 = h.reshape(_NW, nch, _CHUNK)
    r3 = r.reshape(_NW, nch, _CHUNK)
    t3 = t.reshape(_NW, nch, _CHUNK)

    ent_h, ent_t = lax.optimization_barrier((entity_embed, entity_embed))

    mesh = plsc.VectorSubcoreMesh(core_axis_name="c", subcore_axis_name="s")

    @functools.partial(
        pl.kernel,
        out_type=jax.ShapeDtypeStruct((B,), jnp.float32),
        mesh=mesh,
        compiler_params=pltpu.CompilerParams(
            needs_layout_passes=False, use_tc_tiling_on_sc=False),
        scratch_types=[
            pltpu.VMEM((nch, _CHUNK), jnp.int32),
            pltpu.VMEM((nch, _CHUNK), jnp.int32),
            pltpu.VMEM((nch, _CHUNK), jnp.int32),
            pltpu.VMEM((bpw, D), jnp.float32),
            pltpu.VMEM((bpw, D), jnp.float32),
            pltpu.VMEM((bpw, D), jnp.float32),
            pltpu.VMEM((bpw,), jnp.float32),
            pltpu.SemaphoreType.DMA,
        ],
    )
    def run(h_hbm, r_hbm, t_hbm, enth_hbm, entt_hbm, rel_hbm, out_hbm,
            hi, ri, ti, hv, rv, tv, ov, sem):
        wid = lax.axis_index("s") * _NC + lax.axis_index("c")

        pltpu.sync_copy(h_hbm.at[wid], hi)
        pltpu.sync_copy(r_hbm.at[wid], ri)
        pltpu.sync_copy(t_hbm.at[wid], ti)

        copies = []
        for c in range(nch):
            dst = pl.ds(c * _CHUNK, _CHUNK)
            copies.append(pltpu.async_copy(enth_hbm.at[hi.at[c]], hv.at[dst], sem))
            copies.append(pltpu.async_copy(entt_hbm.at[ti.at[c]], tv.at[dst], sem))
            copies.append(pltpu.async_copy(rel_hbm.at[ri.at[c]], rv.at[dst], sem))
        for cp in copies:
            cp.wait()

        def group(g, carry):
            rows = g * _L + lax.iota(jnp.int32, _L)
            h2 = jnp.zeros((_L,), jnp.float32)
            t2 = jnp.zeros((_L,), jnp.float32)
            for j in range(D):
                cj = jnp.full((_L,), j, jnp.int32)
                hj = plsc.load_gather(hv, [rows, cj])
                tj = plsc.load_gather(tv, [rows, cj])
                h2 = h2 + hj * hj
                t2 = t2 + tj * tj
            ih = _rsqrt_newton(h2)
            it = _rsqrt_newton(t2)
            d = jnp.zeros((_L,), jnp.float32)
            for j in range(D):
                cj = jnp.full((_L,), j, jnp.int32)
                hj = plsc.load_gather(hv, [rows, cj])
                rj = plsc.load_gather(rv, [rows, cj])
                tj = plsc.load_gather(tv, [rows, cj])
                d = d + jnp.abs(hj * ih + rj - tj * it)
            ov[pl.ds(pl.multiple_of(g * _L, _L), _L)] = -d
            return carry

        lax.fori_loop(0, ngrp, group, 0)
        pltpu.sync_copy(ov, out_hbm.at[pl.ds(wid * bpw, bpw)])

    return run(h3, r3, t3, ent_h, ent_t, relation_embed)


# speedup vs baseline: 7.6795x; 2.2339x over previous
import functools

import jax
import jax.numpy as jnp
from jax import lax
from jax.experimental import pallas as pl
from jax.experimental.pallas import tpu as pltpu
from jax.experimental.pallas import tpu_sc as plsc

_NC = 2
_NS = 16
_NW = _NC * _NS
_L = 16
_CHUNK = 128


def _rsqrt_newton(x):
    x = jnp.maximum(x, jnp.float32(1e-24))
    i = lax.bitcast_convert_type(x, jnp.int32)
    i = jnp.int32(0x5F3759DF) - lax.shift_right_arithmetic(i, 1)
    y = lax.bitcast_convert_type(i, jnp.float32)
    for _ in range(3):
        y = y * (jnp.float32(1.5) - jnp.float32(0.5) * x * y * y)
    return y


def kernel(h, r, t, entity_embed, relation_embed):
    B = h.shape[0]
    D = entity_embed.shape[1]
    bpw = B // _NW
    nch = bpw // _CHUNK
    ngrp = bpw // _L

    h3 = h.reshape(_NW, nch, _CHUNK)
    r3 = r.reshape(_NW, nch, _CHUNK)
    t3 = t.reshape(_NW, nch, _CHUNK)

    mesh = plsc.VectorSubcoreMesh(core_axis_name="c", subcore_axis_name="s")

    @functools.partial(
        pl.kernel,
        out_type=jax.ShapeDtypeStruct((B,), jnp.float32),
        mesh=mesh,
        compiler_params=pltpu.CompilerParams(
            needs_layout_passes=False, use_tc_tiling_on_sc=False),
        scratch_types=[
            pltpu.VMEM((nch, _CHUNK), jnp.int32),
            pltpu.VMEM((nch, _CHUNK), jnp.int32),
            pltpu.VMEM((nch, _CHUNK), jnp.int32),
            pltpu.VMEM((bpw, D), jnp.float32),
            pltpu.VMEM((bpw, D), jnp.float32),
            pltpu.VMEM((bpw, D), jnp.float32),
            pltpu.VMEM((bpw,), jnp.float32),
            pltpu.SemaphoreType.DMA,
        ],
    )
    def run(h_hbm, r_hbm, t_hbm, ent_hbm, rel_hbm, out_hbm,
            hi, ri, ti, hv, rv, tv, ov, sem):
        wid = lax.axis_index("s") * _NC + lax.axis_index("c")

        pltpu.sync_copy(h_hbm.at[wid], hi)
        pltpu.sync_copy(r_hbm.at[wid], ri)
        pltpu.sync_copy(t_hbm.at[wid], ti)

        copies = []
        for c in range(nch):
            dst = pl.ds(c * _CHUNK, _CHUNK)
            copies.append(pltpu.async_copy(ent_hbm.at[hi.at[c]], hv.at[dst], sem))
            copies.append(pltpu.async_copy(ent_hbm.at[ti.at[c]], tv.at[dst], sem))
            copies.append(pltpu.async_copy(rel_hbm.at[ri.at[c]], rv.at[dst], sem))
        for cp in copies:
            cp.wait()

        def group(g, carry):
            rows = g * _L + lax.iota(jnp.int32, _L)
            h2 = jnp.zeros((_L,), jnp.float32)
            t2 = jnp.zeros((_L,), jnp.float32)
            for j in range(D):
                cj = jnp.full((_L,), j, jnp.int32)
                hj = plsc.load_gather(hv, [rows, cj])
                tj = plsc.load_gather(tv, [rows, cj])
                h2 = h2 + hj * hj
                t2 = t2 + tj * tj
            ih = _rsqrt_newton(h2)
            it = _rsqrt_newton(t2)
            d = jnp.zeros((_L,), jnp.float32)
            for j in range(D):
                cj = jnp.full((_L,), j, jnp.int32)
                hj = plsc.load_gather(hv, [rows, cj])
                rj = plsc.load_gather(rv, [rows, cj])
                tj = plsc.load_gather(tv, [rows, cj])
                d = d + jnp.abs(hj * ih + rj - tj * it)
            ov[pl.ds(pl.multiple_of(g * _L, _L), _L)] = -d
            return carry

        lax.fori_loop(0, ngrp, group, 0)
        pltpu.sync_copy(ov, out_hbm.at[pl.ds(wid * bpw, bpw)])

    out = run(h3, r3, t3, entity_embed, relation_embed)
    probe = jnp.take(entity_embed, h, axis=0)
    return out + jnp.float32(0.0) * jnp.sum(probe)


# speedup vs baseline: 7.7544x; 1.0097x over previous
import functools

import jax
import jax.numpy as jnp
from jax import lax
from jax.experimental import pallas as pl
from jax.experimental.pallas import tpu as pltpu
from jax.experimental.pallas import tpu_sc as plsc

_NC = 2
_NS = 16
_NW = _NC * _NS
_L = 16
_CHUNK = 128


def _rsqrt_newton(x):
    x = jnp.maximum(x, jnp.float32(1e-24))
    i = lax.bitcast_convert_type(x, jnp.int32)
    i = jnp.int32(0x5F3759DF) - lax.shift_right_arithmetic(i, 1)
    y = lax.bitcast_convert_type(i, jnp.float32)
    for _ in range(3):
        y = y * (jnp.float32(1.5) - jnp.float32(0.5) * x * y * y)
    return y


def kernel(h, r, t, entity_embed, relation_embed):
    B = h.shape[0]
    D = entity_embed.shape[1]
    bpw = B // _NW
    nch = bpw // _CHUNK
    ngrp = bpw // _L

    h3 = h.reshape(_NW, nch, _CHUNK)
    r3 = r.reshape(_NW, nch, _CHUNK)
    t3 = t.reshape(_NW, nch, _CHUNK)

    mesh = plsc.VectorSubcoreMesh(core_axis_name="c", subcore_axis_name="s")

    @functools.partial(
        pl.kernel,
        out_type=jax.ShapeDtypeStruct((B,), jnp.float32),
        mesh=mesh,
        compiler_params=pltpu.CompilerParams(
            needs_layout_passes=False, use_tc_tiling_on_sc=False),
        scratch_types=[
            pltpu.VMEM((nch, _CHUNK), jnp.int32),
            pltpu.VMEM((nch, _CHUNK), jnp.int32),
            pltpu.VMEM((nch, _CHUNK), jnp.int32),
            pltpu.VMEM((bpw, D), jnp.float32),
            pltpu.VMEM((bpw, D), jnp.float32),
            pltpu.VMEM((bpw, D), jnp.float32),
            pltpu.VMEM((bpw,), jnp.float32),
            pltpu.SemaphoreType.DMA,
        ],
    )
    def run(h_hbm, r_hbm, t_hbm, ent_hbm, rel_hbm, out_hbm,
            hi, ri, ti, hv, rv, tv, ov, sem):
        wid = lax.axis_index("s") * _NC + lax.axis_index("c")

        pltpu.sync_copy(h_hbm.at[wid], hi)
        pltpu.sync_copy(r_hbm.at[wid], ri)
        pltpu.sync_copy(t_hbm.at[wid], ti)

        copies = []
        for c in range(nch):
            dst = pl.ds(c * _CHUNK, _CHUNK)
            copies.append(pltpu.async_copy(ent_hbm.at[hi.at[c]], hv.at[dst], sem))
            copies.append(pltpu.async_copy(ent_hbm.at[ti.at[c]], tv.at[dst], sem))
            copies.append(pltpu.async_copy(rel_hbm.at[ri.at[c]], rv.at[dst], sem))
        for cp in copies:
            cp.wait()

        def group(g, carry):
            rows = g * _L + lax.iota(jnp.int32, _L)
            h2 = jnp.zeros((_L,), jnp.float32)
            t2 = jnp.zeros((_L,), jnp.float32)
            for j in range(D):
                cj = jnp.full((_L,), j, jnp.int32)
                hj = plsc.load_gather(hv, [rows, cj])
                tj = plsc.load_gather(tv, [rows, cj])
                h2 = h2 + hj * hj
                t2 = t2 + tj * tj
            ih = _rsqrt_newton(h2)
            it = _rsqrt_newton(t2)
            d = jnp.zeros((_L,), jnp.float32)
            for j in range(D):
                cj = jnp.full((_L,), j, jnp.int32)
                hj = plsc.load_gather(hv, [rows, cj])
                rj = plsc.load_gather(rv, [rows, cj])
                tj = plsc.load_gather(tv, [rows, cj])
                d = d + jnp.abs(hj * ih + rj - tj * it)
            ov[pl.ds(pl.multiple_of(g * _L, _L), _L)] = -d
            return carry

        lax.fori_loop(0, ngrp, group, 0)
        pltpu.sync_copy(ov, out_hbm.at[pl.ds(wid * bpw, bpw)])

    return run(h3, r3, t3, entity_embed, relation_embed)
